# Initial kernel scaffold; baseline (speedup 1.0000x reference)
import jax
import jax.numpy as jnp
from jax.experimental import pallas as pl


def kernel(x, edge_index, ln_gamma, ln_beta, W1, b1, Wh, bh, W2, b2, Wout, bout):
    raise NotImplementedError("write your pallas kernel here")


# speedup vs baseline: 7.1758x; 7.1758x over previous
import functools

import jax
import jax.numpy as jnp
from jax import lax
from jax.experimental import pallas as pl
from jax.experimental.pallas import tpu as pltpu
from jax.experimental.pallas import tpu_sc as plsc

NN = 10000
DD = 128
DOUT = 64
EE = 320000

NC = 2
NS = 16
NWORK = NC * NS
CHUNK = 128
CPT = 80
CPW = 40
NP = 10112
RPT = NP // NS
DEGW = 128
RB = 2000


def _fill(buf, rows, width, value):
    def row(r, _):
        for c in range(width // 16):
            buf[r, pl.ds(c * 16, 16)] = jnp.full((16,), value, jnp.float32)
        return 0
    lax.fori_loop(0, rows, row, 0)


def _spmem_zero_init(shared, buf, sid, width):
    _fill(buf, 128, width, 0.0)
    base = sid * RPT
    for k in range(4):
        pltpu.sync_copy(buf, shared.at[pl.ds(base + 128 * k, 128)])
    pltpu.sync_copy(buf.at[pl.ds(0, RPT - 512)],
                    shared.at[pl.ds(base + 512, RPT - 512)])


def _sc_degree_body(dst_hbm, out_hbm, deg_sh, dst_v, ones_v):
    c = lax.axis_index("c")
    s = lax.axis_index("s")
    wid = c * NS + s
    _spmem_zero_init(deg_sh, ones_v, s, DEGW)
    _fill(ones_v, CHUNK, DEGW, 1.0)
    pltpu.sync_copy(dst_hbm.at[wid], dst_v)
    plsc.subcore_barrier()

    def body(i, _):
        pltpu.sync_copy(ones_v, deg_sh.at[dst_v.at[i]], add=True)
        return 0
    lax.fori_loop(0, CPT, body, 0)

    plsc.subcore_barrier()
    pltpu.sync_copy(deg_sh.at[pl.ds(s * RPT, RPT)],
                    out_hbm.at[c].at[pl.ds(s * RPT, RPT)])


def _sc_aggregate_body(g_hbm, src_hbm, dst_hbm, out_hbm,
                       acc_sh, src_v, dst_v, buf0, buf1, sem0, sem1):
    c = lax.axis_index("c")
    s = lax.axis_index("s")
    wid = c * NS + s
    _spmem_zero_init(acc_sh, buf0, s, DD)
    plsc.subcore_barrier()

    for p in range(CPT // CPW):
        pltpu.sync_copy(src_hbm.at[wid].at[pl.ds(p * CPW, CPW)], src_v)
        pltpu.sync_copy(dst_hbm.at[wid].at[pl.ds(p * CPW, CPW)], dst_v)
        pltpu.async_copy(g_hbm.at[src_v.at[0]], buf0, sem0)
        pltpu.async_copy(g_hbm.at[src_v.at[1]], buf1, sem1)

        def body(i, _):
            ch = 2 * i
            pltpu.make_async_copy(g_hbm.at[src_v.at[ch]], buf0, sem0).wait()
            pltpu.sync_copy(buf0, acc_sh.at[dst_v.at[ch]], add=True)
            pltpu.async_copy(g_hbm.at[src_v.at[ch + 2]], buf0, sem0)
            pltpu.make_async_copy(g_hbm.at[src_v.at[ch + 1]], buf1, sem1).wait()
            pltpu.sync_copy(buf1, acc_sh.at[dst_v.at[ch + 1]], add=True)
            pltpu.async_copy(g_hbm.at[src_v.at[ch + 3]], buf1, sem1)
            return 0
        lax.fori_loop(0, (CPW - 2) // 2, body, 0)
        pltpu.make_async_copy(g_hbm.at[src_v.at[CPW - 2]], buf0, sem0).wait()
        pltpu.sync_copy(buf0, acc_sh.at[dst_v.at[CPW - 2]], add=True)
        pltpu.make_async_copy(g_hbm.at[src_v.at[CPW - 1]], buf1, sem1).wait()
        pltpu.sync_copy(buf1, acc_sh.at[dst_v.at[CPW - 1]], add=True)

    plsc.subcore_barrier()
    pltpu.sync_copy(acc_sh.at[pl.ds(s * RPT, RPT)],
                    out_hbm.at[c].at[pl.ds(s * RPT, RPT)])


_SC_MESH = plsc.VectorSubcoreMesh(core_axis_name="c", subcore_axis_name="s",
                                  num_cores=NC, num_subcores=NS)

_sc_degree = pl.kernel(
    _sc_degree_body,
    out_type=jax.ShapeDtypeStruct((NC, NP, DEGW), jnp.float32),
    mesh=_SC_MESH,
    scratch_types=[
        pltpu.VMEM_SHARED((NP, DEGW), jnp.float32),
        pltpu.VMEM((CPT, CHUNK), jnp.int32),
        pltpu.VMEM((CHUNK, DEGW), jnp.float32),
    ],
)

_sc_aggregate = pl.kernel(
    _sc_aggregate_body,
    out_type=jax.ShapeDtypeStruct((NC, NP, DD), jnp.float32),
    mesh=_SC_MESH,
    scratch_types=[
        pltpu.VMEM_SHARED((NP, DD), jnp.float32),
        pltpu.VMEM((CPW, CHUNK), jnp.int32),
        pltpu.VMEM((CPW, CHUNK), jnp.int32),
        pltpu.VMEM((CHUNK, DD), jnp.float32),
        pltpu.VMEM((CHUNK, DD), jnp.float32),
        pltpu.SemaphoreType.DMA,
        pltpu.SemaphoreType.DMA,
    ],
)


def _dis_of(deg_ref):
    d = deg_ref[0, :, 0] + deg_ref[1, :, 0] + 1.0
    return lax.rsqrt(d)[:, None]


def _pre_body(x_ref, ga_ref, be_ref, w_ref, deg_ref, g_out):
    x = x_ref[...]
    mu = jnp.mean(x, axis=1, keepdims=True)
    xc = x - mu
    var = jnp.mean(xc * xc, axis=1, keepdims=True)
    ln = xc * lax.rsqrt(var + 1e-5) * ga_ref[...] + be_ref[...]
    u = jnp.dot(ln, w_ref[...], preferred_element_type=jnp.float32)
    g_out[...] = u * _dis_of(deg_ref)


def _mid_body(acc_ref, g_ref, deg_ref, b_ref, w_ref, g_out):
    dis = _dis_of(deg_ref)
    h = dis * (acc_ref[0] + acc_ref[1] + g_ref[...]) + b_ref[...]
    h = jnp.maximum(h, 0.0)
    u = jnp.dot(h, w_ref[...], preferred_element_type=jnp.float32)
    g_out[...] = u * dis


def _post_body(acc_ref, g_ref, deg_ref, b_ref, wo_ref, bo_ref, out_ref, h_out):
    dis = _dis_of(deg_ref)
    h = dis * (acc_ref[0] + acc_ref[1] + g_ref[...]) + b_ref[...]
    h_out[...] = h
    out_ref[...] = jnp.dot(h, wo_ref[...],
                           preferred_element_type=jnp.float32) + bo_ref[...]


_ROWS = pl.BlockSpec((RB, DD), lambda i: (i, 0))
_VEC = pl.BlockSpec((1, DD), lambda i: (0, 0))
_MAT = pl.BlockSpec((DD, DD), lambda i: (0, 0))
_DEG = pl.BlockSpec((NC, RB, DEGW), lambda i: (0, i, 0))
_ACC = pl.BlockSpec((NC, RB, DD), lambda i: (0, i, 0))
_GRID = (NN // RB,)

_tc_pre = pl.pallas_call(
    _pre_body,
    grid=_GRID,
    in_specs=[_ROWS, _VEC, _VEC, _MAT, _DEG],
    out_specs=_ROWS,
    out_shape=jax.ShapeDtypeStruct((NN, DD), jnp.float32),
)

_tc_mid = pl.pallas_call(
    _mid_body,
    grid=_GRID,
    in_specs=[_ACC, _ROWS, _DEG, _VEC, _MAT],
    out_specs=_ROWS,
    out_shape=jax.ShapeDtypeStruct((NN, DD), jnp.float32),
)

_tc_post = pl.pallas_call(
    _post_body,
    grid=_GRID,
    in_specs=[_ACC, _ROWS, _DEG, _VEC,
              pl.BlockSpec((DD, DOUT), lambda i: (0, 0)),
              pl.BlockSpec((1, DOUT), lambda i: (0, 0))],
    out_specs=[pl.BlockSpec((RB, DOUT), lambda i: (i, 0)), _ROWS],
    out_shape=[jax.ShapeDtypeStruct((NN, DOUT), jnp.float32),
               jax.ShapeDtypeStruct((NN, DD), jnp.float32)],
)


def kernel(x, edge_index, ln_gamma, ln_beta, W1, b1, Wh, bh, W2, b2, Wout, bout):
    ep = NWORK * CPT * CHUNK - EE
    src = jnp.concatenate([edge_index[0], jnp.zeros((ep,), jnp.int32)])
    dst = jnp.concatenate([edge_index[1], jnp.full((ep,), NN, jnp.int32)])
    src = src.reshape(NWORK, CPT, CHUNK)
    dst = dst.reshape(NWORK, CPT, CHUNK)

    gamma = ln_gamma.reshape(1, DD)
    beta = ln_beta.reshape(1, DD)

    deg = _sc_degree(dst)
    g_a = _tc_pre(x, gamma, beta, W1, deg)
    acc_a = _sc_aggregate(g_a, src, dst)
    g_b = _tc_mid(acc_a, g_a, deg, b1.reshape(1, DD), Wh)
    acc_b = _sc_aggregate(g_b, src, dst)
    g_c = _tc_mid(acc_b, g_b, deg, bh.reshape(1, DD), W2)
    acc_c = _sc_aggregate(g_c, src, dst)
    out, h = _tc_post(acc_c, g_c, deg, b2.reshape(1, DD),
                      Wout, bout.reshape(1, DOUT))
    return (out, h)


# speedup vs baseline: 17.8972x; 2.4941x over previous
import functools

import jax
import jax.numpy as jnp
from jax import lax
from jax.experimental import pallas as pl
from jax.experimental.pallas import tpu as pltpu
from jax.experimental.pallas import tpu_sc as plsc

NN = 10000
DD = 128
DH = 64
DOUT = 64
EE = 320000

NC = 2
NS = 16
CHUNK = 128
CPT = 160
CPW = 40
NP = 10112
RPT = NP // NS
DEGW = 16
RB = 2000

_SC_PARAMS = pltpu.CompilerParams(use_tc_tiling_on_sc=False)
_SC_MESH = plsc.VectorSubcoreMesh(core_axis_name="c", subcore_axis_name="s",
                                  num_cores=NC, num_subcores=NS)


def _fill(buf, rows, width, value):
    def row(r, _):
        for c in range(width // 16):
            buf[r, pl.ds(c * 16, 16)] = jnp.full((16,), value, jnp.float32)
        return 0
    lax.fori_loop(0, rows, row, 0)


def _spmem_zero_init(shared, buf, sid, width):
    _fill(buf, 128, width, 0.0)
    base = sid * RPT
    for k in range(4):
        pltpu.sync_copy(buf, shared.at[pl.ds(base + 128 * k, 128)])
    pltpu.sync_copy(buf.at[pl.ds(0, RPT - 512)],
                    shared.at[pl.ds(base + 512, RPT - 512)])


def _sc_degree_body(dst_hbm, out_hbm, deg_sh, dst_v, ones_v):
    c = lax.axis_index("c")
    s = lax.axis_index("s")
    wid = c * NS + s
    _spmem_zero_init(deg_sh, ones_v, s, DEGW)
    _fill(ones_v, CHUNK, DEGW, 1.0)
    pltpu.sync_copy(dst_hbm.at[wid], dst_v)
    plsc.subcore_barrier()

    def body(i, _):
        pltpu.sync_copy(ones_v, deg_sh.at[dst_v.at[i]], add=True)
        return 0
    lax.fori_loop(0, CPT // 2, body, 0)

    plsc.subcore_barrier()
    pltpu.sync_copy(deg_sh.at[pl.ds(s * RPT, RPT)],
                    out_hbm.at[c].at[pl.ds(s * RPT, RPT)])


def _sc_aggregate_body(g_hbm, src_hbm, dst_hbm, out_hbm,
                       g_sh, acc_sh, src_v, dst_v, buf0, buf1, sem0, sem1):
    c = lax.axis_index("c")
    s = lax.axis_index("s")
    _spmem_zero_init(acc_sh, buf0, s, DH)
    start = jnp.minimum(s * RPT, NN - RPT)
    pltpu.sync_copy(g_hbm.at[c].at[pl.ds(start, RPT)],
                    g_sh.at[pl.ds(start, RPT)])
    plsc.subcore_barrier()

    for p in range(CPT // CPW):
        pltpu.sync_copy(src_hbm.at[s].at[pl.ds(p * CPW, CPW)], src_v)
        pltpu.sync_copy(dst_hbm.at[s].at[pl.ds(p * CPW, CPW)], dst_v)
        pltpu.async_copy(g_sh.at[src_v.at[0]], buf0, sem0)
        pltpu.async_copy(g_sh.at[src_v.at[1]], buf1, sem1)

        def body(i, _):
            ch = 2 * i
            pltpu.make_async_copy(g_sh.at[src_v.at[ch]], buf0, sem0).wait()
            pltpu.sync_copy(buf0, acc_sh.at[dst_v.at[ch]], add=True)
            pltpu.async_copy(g_sh.at[src_v.at[ch + 2]], buf0, sem0)
            pltpu.make_async_copy(g_sh.at[src_v.at[ch + 1]], buf1, sem1).wait()
            pltpu.sync_copy(buf1, acc_sh.at[dst_v.at[ch + 1]], add=True)
            pltpu.async_copy(g_sh.at[src_v.at[ch + 3]], buf1, sem1)
            return 0
        lax.fori_loop(0, (CPW - 2) // 2, body, 0)
        pltpu.make_async_copy(g_sh.at[src_v.at[CPW - 2]], buf0, sem0).wait()
        pltpu.sync_copy(buf0, acc_sh.at[dst_v.at[CPW - 2]], add=True)
        pltpu.make_async_copy(g_sh.at[src_v.at[CPW - 1]], buf1, sem1).wait()
        pltpu.sync_copy(buf1, acc_sh.at[dst_v.at[CPW - 1]], add=True)

    plsc.subcore_barrier()
    pltpu.sync_copy(acc_sh.at[pl.ds(s * RPT, RPT)],
                    out_hbm.at[c].at[pl.ds(s * RPT, RPT)])


_sc_degree = pl.kernel(
    _sc_degree_body,
    out_type=jax.ShapeDtypeStruct((NC, NP, DEGW), jnp.float32),
    mesh=_SC_MESH,
    scratch_types=[
        pltpu.VMEM_SHARED((NP, DEGW), jnp.float32),
        pltpu.VMEM((CPT // 2, CHUNK), jnp.int32),
        pltpu.VMEM((CHUNK, DEGW), jnp.float32),
    ],
    compiler_params=_SC_PARAMS,
)

_sc_aggregate = pl.kernel(
    _sc_aggregate_body,
    out_type=jax.ShapeDtypeStruct((NC, NP, DH), jnp.float32),
    mesh=_SC_MESH,
    scratch_types=[
        pltpu.VMEM_SHARED((NP, DH), jnp.float32),
        pltpu.VMEM_SHARED((NP, DH), jnp.float32),
        pltpu.VMEM((CPW, CHUNK), jnp.int32),
        pltpu.VMEM((CPW, CHUNK), jnp.int32),
        pltpu.VMEM((CHUNK, DH), jnp.float32),
        pltpu.VMEM((CHUNK, DH), jnp.float32),
        pltpu.SemaphoreType.DMA,
        pltpu.SemaphoreType.DMA,
    ],
    compiler_params=_SC_PARAMS,
)


def _dis_of(deg_ref):
    d = deg_ref[0, :, 0] + deg_ref[1, :, 0] + 1.0
    return lax.rsqrt(d)[:, None]


def _pre_body(x_ref, ga_ref, be_ref, w_ref, deg_ref, g_out):
    x = x_ref[...]
    mu = jnp.mean(x, axis=1, keepdims=True)
    xc = x - mu
    var = jnp.mean(xc * xc, axis=1, keepdims=True)
    ln = xc * lax.rsqrt(var + 1e-5) * ga_ref[...] + be_ref[...]
    u = jnp.dot(ln, w_ref[...], preferred_element_type=jnp.float32)
    g = u * _dis_of(deg_ref)
    g_out[0] = g[:, :DH]
    g_out[1] = g[:, DH:]


def _mid_body(acc_ref, g_ref, deg_ref, b_ref, w_ref, g_out):
    dis = _dis_of(deg_ref)
    h0 = jnp.maximum(dis * (acc_ref[0] + g_ref[0]) + b_ref[0], 0.0)
    h1 = jnp.maximum(dis * (acc_ref[1] + g_ref[1]) + b_ref[1], 0.0)
    u = (jnp.dot(h0, w_ref[0], preferred_element_type=jnp.float32)
         + jnp.dot(h1, w_ref[1], preferred_element_type=jnp.float32))
    g = u * dis
    g_out[0] = g[:, :DH]
    g_out[1] = g[:, DH:]


def _post_body(acc_ref, g_ref, deg_ref, b_ref, wo_ref, bo_ref, out_ref, h_out):
    dis = _dis_of(deg_ref)
    h0 = dis * (acc_ref[0] + g_ref[0]) + b_ref[0]
    h1 = dis * (acc_ref[1] + g_ref[1]) + b_ref[1]
    h_out[...] = jnp.concatenate([h0, h1], axis=1)
    out_ref[...] = (jnp.dot(h0, wo_ref[0], preferred_element_type=jnp.float32)
                    + jnp.dot(h1, wo_ref[1], preferred_element_type=jnp.float32)
                    + bo_ref[...])


_ROWS = pl.BlockSpec((RB, DD), lambda i: (i, 0))
_HALVES = pl.BlockSpec((NC, RB, DH), lambda i: (0, i, 0))
_BVEC = pl.BlockSpec((NC, 1, DH), lambda i: (0, 0, 0))
_VEC = pl.BlockSpec((1, DD), lambda i: (0, 0))
_MAT = pl.BlockSpec((DD, DD), lambda i: (0, 0))
_WSPLIT = pl.BlockSpec((NC, DH, DD), lambda i: (0, 0, 0))
_DEG = pl.BlockSpec((NC, RB, DEGW), lambda i: (0, i, 0))
_GRID = (NN // RB,)

_tc_pre = pl.pallas_call(
    _pre_body,
    grid=_GRID,
    in_specs=[_ROWS, _VEC, _VEC, _MAT, _DEG],
    out_specs=_HALVES,
    out_shape=jax.ShapeDtypeStruct((NC, NN, DH), jnp.float32),
)

_tc_mid = pl.pallas_call(
    _mid_body,
    grid=_GRID,
    in_specs=[_HALVES, _HALVES, _DEG, _BVEC, _WSPLIT],
    out_specs=_HALVES,
    out_shape=jax.ShapeDtypeStruct((NC, NN, DH), jnp.float32),
)

_tc_post = pl.pallas_call(
    _post_body,
    grid=_GRID,
    in_specs=[_HALVES, _HALVES, _DEG, _BVEC,
              pl.BlockSpec((NC, DH, DOUT), lambda i: (0, 0, 0)),
              pl.BlockSpec((1, DOUT), lambda i: (0, 0))],
    out_specs=[pl.BlockSpec((RB, DOUT), lambda i: (i, 0)), _ROWS],
    out_shape=[jax.ShapeDtypeStruct((NN, DOUT), jnp.float32),
               jax.ShapeDtypeStruct((NN, DD), jnp.float32)],
)


def kernel(x, edge_index, ln_gamma, ln_beta, W1, b1, Wh, bh, W2, b2, Wout, bout):
    ep = NS * CPT * CHUNK - EE
    src = jnp.concatenate([edge_index[0], jnp.zeros((ep,), jnp.int32)])
    dst = jnp.concatenate([edge_index[1], jnp.full((ep,), NN, jnp.int32)])
    src = src.reshape(NS, CPT, CHUNK)
    dst = dst.reshape(NS, CPT, CHUNK)
    dst_deg = dst.reshape(NC * NS, CPT // 2, CHUNK)

    gamma = ln_gamma.reshape(1, DD)
    beta = ln_beta.reshape(1, DD)

    def bsplit(b):
        return b.reshape(NC, 1, DH)

    def wsplit(w):
        return w.reshape(NC, DH, w.shape[1])

    deg = _sc_degree(dst_deg)
    g_a = _tc_pre(x, gamma, beta, W1, deg)
    acc_a = _sc_aggregate(g_a, src, dst)
    g_b = _tc_mid(acc_a, g_a, deg, bsplit(b1), wsplit(Wh))
    acc_b = _sc_aggregate(g_b, src, dst)
    g_c = _tc_mid(acc_b, g_b, deg, bsplit(bh), wsplit(W2))
    acc_c = _sc_aggregate(g_c, src, dst)
    out, h = _tc_post(acc_c, g_c, deg, bsplit(b2),
                      wsplit(Wout), bout.reshape(1, DOUT))
    return (out, h)


# speedup vs baseline: 20.9772x; 1.1721x over previous
import functools

import jax
import jax.numpy as jnp
from jax import lax
from jax.experimental import pallas as pl
from jax.experimental.pallas import tpu as pltpu
from jax.experimental.pallas import tpu_sc as plsc

NN = 10000
DD = 128
DH = 64
DOUT = 64
EE = 320000

NC = 2
NS = 16
CHUNK = 128
CPT = 160
CPW = 40
NP = 10112
RPT = NP // NS
DEGW = 16
RB = 2000

_SC_PARAMS = pltpu.CompilerParams(use_tc_tiling_on_sc=False)
_SC_MESH = plsc.VectorSubcoreMesh(core_axis_name="c", subcore_axis_name="s",
                                  num_cores=NC, num_subcores=NS)


def _fill(buf, rows, width, value):
    def row(r, _):
        for c in range(width // 16):
            buf[r, pl.ds(c * 16, 16)] = jnp.full((16,), value, jnp.float32)
        return 0
    lax.fori_loop(0, rows, row, 0)


def _spmem_zero_init(shared, buf, sid, width):
    _fill(buf, 128, width, 0.0)
    base = sid * RPT
    for k in range(4):
        pltpu.sync_copy(buf, shared.at[pl.ds(base + 128 * k, 128)])
    pltpu.sync_copy(buf.at[pl.ds(0, RPT - 512)],
                    shared.at[pl.ds(base + 512, RPT - 512)])


def _sc_degree_body(dst_hbm, out_hbm, deg_sh, dst_v, ones_v):
    c = lax.axis_index("c")
    s = lax.axis_index("s")
    wid = c * NS + s
    _spmem_zero_init(deg_sh, ones_v, s, DEGW)
    _fill(ones_v, CHUNK, DEGW, 1.0)
    pltpu.sync_copy(dst_hbm.at[wid], dst_v)
    plsc.subcore_barrier()

    def body(i, _):
        pltpu.sync_copy(ones_v, deg_sh.at[dst_v.at[i]], add=True)
        return 0
    lax.fori_loop(0, CPT // 2, body, 0)

    plsc.subcore_barrier()
    pltpu.sync_copy(deg_sh.at[pl.ds(s * RPT, RPT)],
                    out_hbm.at[c].at[pl.ds(s * RPT, RPT)])


def _sc_aggregate_body(g_hbm, src_hbm, dst_hbm, out_hbm,
                       g_sh, acc_sh, src_v, dst_v,
                       buf0, buf1, buf2, buf3, gsem, ssem):
    c = lax.axis_index("c")
    s = lax.axis_index("s")
    _spmem_zero_init(acc_sh, buf0, s, DH)
    start = jnp.minimum(s * RPT, NN - RPT)
    pltpu.sync_copy(g_hbm.at[c].at[pl.ds(start, RPT)],
                    g_sh.at[pl.ds(start, RPT)])
    plsc.subcore_barrier()

    bufs = (buf0, buf1, buf2, buf3)
    for p in range(CPT // CPW):
        pltpu.sync_copy(src_hbm.at[s].at[pl.ds(p * CPW, CPW)], src_v)
        pltpu.sync_copy(dst_hbm.at[s].at[pl.ds(p * CPW, CPW)], dst_v)
        pltpu.async_copy(g_sh.at[src_v.at[0]], bufs[0], gsem)
        pltpu.async_copy(g_sh.at[src_v.at[1]], bufs[1], gsem)

        def body(i, _):
            for j in range(4):
                q = 4 * i + j
                pltpu.make_async_copy(g_sh.at[src_v.at[q]], bufs[j],
                                      gsem).wait()
                pltpu.async_copy(bufs[j], acc_sh.at[dst_v.at[q]], ssem,
                                 add=True)
                if j < 2:
                    @pl.when(i >= 1)
                    def _():
                        pltpu.make_async_copy(bufs[0], acc_sh.at[dst_v.at[0]],
                                              ssem).wait()
                else:
                    pltpu.make_async_copy(bufs[0], acc_sh.at[dst_v.at[0]],
                                          ssem).wait()
                jn = (j + 2) % 4
                if j < 2:
                    pltpu.async_copy(g_sh.at[src_v.at[q + 2]], bufs[jn], gsem)
                else:
                    @pl.when(i < CPW // 4 - 1)
                    def _():
                        pltpu.async_copy(g_sh.at[src_v.at[q + 2]], bufs[jn],
                                         gsem)
            return 0
        lax.fori_loop(0, CPW // 4, body, 0)
        for _ in range(2):
            pltpu.make_async_copy(bufs[0], acc_sh.at[dst_v.at[0]],
                                  ssem).wait()

    plsc.subcore_barrier()
    pltpu.sync_copy(acc_sh.at[pl.ds(s * RPT, RPT)],
                    out_hbm.at[c].at[pl.ds(s * RPT, RPT)])


_sc_degree = pl.kernel(
    _sc_degree_body,
    out_type=jax.ShapeDtypeStruct((NC, NP, DEGW), jnp.float32),
    mesh=_SC_MESH,
    scratch_types=[
        pltpu.VMEM_SHARED((NP, DEGW), jnp.float32),
        pltpu.VMEM((CPT // 2, CHUNK), jnp.int32),
        pltpu.VMEM((CHUNK, DEGW), jnp.float32),
    ],
    compiler_params=_SC_PARAMS,
)

_sc_aggregate = pl.kernel(
    _sc_aggregate_body,
    out_type=jax.ShapeDtypeStruct((NC, NP, DH), jnp.float32),
    mesh=_SC_MESH,
    scratch_types=[
        pltpu.VMEM_SHARED((NP, DH), jnp.float32),
        pltpu.VMEM_SHARED((NP, DH), jnp.float32),
        pltpu.VMEM((CPW, CHUNK), jnp.int32),
        pltpu.VMEM((CPW, CHUNK), jnp.int32),
        pltpu.VMEM((CHUNK, DH), jnp.float32),
        pltpu.VMEM((CHUNK, DH), jnp.float32),
        pltpu.VMEM((CHUNK, DH), jnp.float32),
        pltpu.VMEM((CHUNK, DH), jnp.float32),
        pltpu.SemaphoreType.DMA,
        pltpu.SemaphoreType.DMA,
    ],
    compiler_params=_SC_PARAMS,
)


def _dis_of(deg_ref):
    d = deg_ref[0, :, 0] + deg_ref[1, :, 0] + 1.0
    return lax.rsqrt(d)[:, None]


def _pre_body(x_ref, ga_ref, be_ref, w_ref, deg_ref, g_out):
    x = x_ref[...]
    mu = jnp.mean(x, axis=1, keepdims=True)
    xc = x - mu
    var = jnp.mean(xc * xc, axis=1, keepdims=True)
    ln = xc * lax.rsqrt(var + 1e-5) * ga_ref[...] + be_ref[...]
    u = jnp.dot(ln, w_ref[...], preferred_element_type=jnp.float32)
    g = u * _dis_of(deg_ref)
    g_out[0] = g[:, :DH]
    g_out[1] = g[:, DH:]


def _mid_body(acc_ref, g_ref, deg_ref, b_ref, w_ref, g_out):
    dis = _dis_of(deg_ref)
    h0 = jnp.maximum(dis * (acc_ref[0] + g_ref[0]) + b_ref[0], 0.0)
    h1 = jnp.maximum(dis * (acc_ref[1] + g_ref[1]) + b_ref[1], 0.0)
    u = (jnp.dot(h0, w_ref[0], preferred_element_type=jnp.float32)
         + jnp.dot(h1, w_ref[1], preferred_element_type=jnp.float32))
    g = u * dis
    g_out[0] = g[:, :DH]
    g_out[1] = g[:, DH:]


def _post_body(acc_ref, g_ref, deg_ref, b_ref, wo_ref, bo_ref, out_ref, h_out):
    dis = _dis_of(deg_ref)
    h0 = dis * (acc_ref[0] + g_ref[0]) + b_ref[0]
    h1 = dis * (acc_ref[1] + g_ref[1]) + b_ref[1]
    h_out[...] = jnp.concatenate([h0, h1], axis=1)
    out_ref[...] = (jnp.dot(h0, wo_ref[0], preferred_element_type=jnp.float32)
                    + jnp.dot(h1, wo_ref[1], preferred_element_type=jnp.float32)
                    + bo_ref[...])


_ROWS = pl.BlockSpec((RB, DD), lambda i: (i, 0))
_HALVES = pl.BlockSpec((NC, RB, DH), lambda i: (0, i, 0))
_BVEC = pl.BlockSpec((NC, 1, DH), lambda i: (0, 0, 0))
_VEC = pl.BlockSpec((1, DD), lambda i: (0, 0))
_MAT = pl.BlockSpec((DD, DD), lambda i: (0, 0))
_WSPLIT = pl.BlockSpec((NC, DH, DD), lambda i: (0, 0, 0))
_DEG = pl.BlockSpec((NC, RB, DEGW), lambda i: (0, i, 0))
_GRID = (NN // RB,)

_tc_pre = pl.pallas_call(
    _pre_body,
    grid=_GRID,
    in_specs=[_ROWS, _VEC, _VEC, _MAT, _DEG],
    out_specs=_HALVES,
    out_shape=jax.ShapeDtypeStruct((NC, NN, DH), jnp.float32),
)

_tc_mid = pl.pallas_call(
    _mid_body,
    grid=_GRID,
    in_specs=[_HALVES, _HALVES, _DEG, _BVEC, _WSPLIT],
    out_specs=_HALVES,
    out_shape=jax.ShapeDtypeStruct((NC, NN, DH), jnp.float32),
)

_tc_post = pl.pallas_call(
    _post_body,
    grid=_GRID,
    in_specs=[_HALVES, _HALVES, _DEG, _BVEC,
              pl.BlockSpec((NC, DH, DOUT), lambda i: (0, 0, 0)),
              pl.BlockSpec((1, DOUT), lambda i: (0, 0))],
    out_specs=[pl.BlockSpec((RB, DOUT), lambda i: (i, 0)), _ROWS],
    out_shape=[jax.ShapeDtypeStruct((NN, DOUT), jnp.float32),
               jax.ShapeDtypeStruct((NN, DD), jnp.float32)],
)


def kernel(x, edge_index, ln_gamma, ln_beta, W1, b1, Wh, bh, W2, b2, Wout, bout):
    ep = NS * CPT * CHUNK - EE
    src = jnp.concatenate([edge_index[0], jnp.zeros((ep,), jnp.int32)])
    dst = jnp.concatenate([edge_index[1], jnp.full((ep,), NN, jnp.int32)])
    src = src.reshape(NS, CPT, CHUNK)
    dst = dst.reshape(NS, CPT, CHUNK)
    dst_deg = dst.reshape(NC * NS, CPT // 2, CHUNK)

    gamma = ln_gamma.reshape(1, DD)
    beta = ln_beta.reshape(1, DD)

    def bsplit(b):
        return b.reshape(NC, 1, DH)

    def wsplit(w):
        return w.reshape(NC, DH, w.shape[1])

    deg = _sc_degree(dst_deg)
    g_a = _tc_pre(x, gamma, beta, W1, deg)
    acc_a = _sc_aggregate(g_a, src, dst)
    g_b = _tc_mid(acc_a, g_a, deg, bsplit(b1), wsplit(Wh))
    acc_b = _sc_aggregate(g_b, src, dst)
    g_c = _tc_mid(acc_b, g_b, deg, bsplit(bh), wsplit(W2))
    acc_c = _sc_aggregate(g_c, src, dst)
    out, h = _tc_post(acc_c, g_c, deg, bsplit(b2),
                      wsplit(Wout), bout.reshape(1, DOUT))
    return (out, h)


# speedup vs baseline: 21.2581x; 1.0134x over previous
import functools

import jax
import jax.numpy as jnp
from jax import lax
from jax.experimental import pallas as pl
from jax.experimental.pallas import tpu as pltpu
from jax.experimental.pallas import tpu_sc as plsc

NN = 10000
DD = 128
DH = 64
DOUT = 64
EE = 320000

NC = 2
NS = 16
CHUNK = 128
CPT = 160
CPW = 40
NP = 10112
RPT = NP // NS
DEGW = 16
RB = 2000

_SC_PARAMS = pltpu.CompilerParams(use_tc_tiling_on_sc=False)
_SC_MESH = plsc.VectorSubcoreMesh(core_axis_name="c", subcore_axis_name="s",
                                  num_cores=NC, num_subcores=NS)


def _fill(buf, rows, width, value):
    def row(r, _):
        for c in range(width // 16):
            buf[r, pl.ds(c * 16, 16)] = jnp.full((16,), value, jnp.float32)
        return 0
    lax.fori_loop(0, rows, row, 0)


def _spmem_zero_init(shared, buf, sid, width):
    _fill(buf, 128, width, 0.0)
    base = sid * RPT
    for k in range(4):
        pltpu.sync_copy(buf, shared.at[pl.ds(base + 128 * k, 128)])
    pltpu.sync_copy(buf.at[pl.ds(0, RPT - 512)],
                    shared.at[pl.ds(base + 512, RPT - 512)])


def _sc_degree_body(dst_hbm, out_hbm, deg_sh, dst_v, ones_v):
    c = lax.axis_index("c")
    s = lax.axis_index("s")
    _spmem_zero_init(deg_sh, ones_v, s, DEGW)
    _fill(ones_v, CHUNK, DEGW, 1.0)
    pltpu.sync_copy(dst_hbm.at[s].at[pl.ds(c * (CPT // 2), CPT // 2)], dst_v)
    plsc.subcore_barrier()

    def body(i, _):
        pltpu.sync_copy(ones_v, deg_sh.at[dst_v.at[i]], add=True)
        return 0
    lax.fori_loop(0, CPT // 2, body, 0)

    plsc.subcore_barrier()
    pltpu.sync_copy(deg_sh.at[pl.ds(s * RPT, RPT)],
                    out_hbm.at[c].at[pl.ds(s * RPT, RPT)])


def _sc_aggregate_body(g_hbm, src_hbm, dst_hbm, out_hbm,
                       g_sh, acc_sh, src_v, dst_v,
                       buf0, buf1, buf2, buf3, gsem, ssem):
    c = lax.axis_index("c")
    s = lax.axis_index("s")
    _spmem_zero_init(acc_sh, buf0, s, DH)
    start = jnp.minimum(s * RPT, NN - RPT)
    pltpu.sync_copy(g_hbm.at[c].at[pl.ds(start, RPT)],
                    g_sh.at[pl.ds(start, RPT)])
    plsc.subcore_barrier()

    bufs = (buf0, buf1, buf2, buf3)
    for p in range(CPT // CPW):
        pltpu.sync_copy(src_hbm.at[s].at[pl.ds(p * CPW, CPW)], src_v)
        pltpu.sync_copy(dst_hbm.at[s].at[pl.ds(p * CPW, CPW)], dst_v)
        pltpu.async_copy(g_sh.at[src_v.at[0]], bufs[0], gsem)
        pltpu.async_copy(g_sh.at[src_v.at[1]], bufs[1], gsem)

        def body(i, _):
            for j in range(4):
                q = 4 * i + j
                pltpu.make_async_copy(g_sh.at[src_v.at[q]], bufs[j],
                                      gsem).wait()
                pltpu.async_copy(bufs[j], acc_sh.at[dst_v.at[q]], ssem,
                                 add=True)
                if j < 2:
                    @pl.when(i >= 1)
                    def _():
                        pltpu.make_async_copy(bufs[0], acc_sh.at[dst_v.at[0]],
                                              ssem).wait()
                else:
                    pltpu.make_async_copy(bufs[0], acc_sh.at[dst_v.at[0]],
                                          ssem).wait()
                jn = (j + 2) % 4
                if j < 2:
                    pltpu.async_copy(g_sh.at[src_v.at[q + 2]], bufs[jn], gsem)
                else:
                    @pl.when(i < CPW // 4 - 1)
                    def _():
                        pltpu.async_copy(g_sh.at[src_v.at[q + 2]], bufs[jn],
                                         gsem)
            return 0
        lax.fori_loop(0, CPW // 4, body, 0)
        for _ in range(2):
            pltpu.make_async_copy(bufs[0], acc_sh.at[dst_v.at[0]],
                                  ssem).wait()

    plsc.subcore_barrier()
    pltpu.sync_copy(acc_sh.at[pl.ds(s * RPT, RPT)],
                    out_hbm.at[c].at[pl.ds(s * RPT, RPT)])


_sc_degree = pl.kernel(
    _sc_degree_body,
    out_type=jax.ShapeDtypeStruct((NC, NP, DEGW), jnp.float32),
    mesh=_SC_MESH,
    scratch_types=[
        pltpu.VMEM_SHARED((NP, DEGW), jnp.float32),
        pltpu.VMEM((CPT // 2, CHUNK), jnp.int32),
        pltpu.VMEM((CHUNK, DEGW), jnp.float32),
    ],
    compiler_params=_SC_PARAMS,
)

_sc_aggregate = pl.kernel(
    _sc_aggregate_body,
    out_type=jax.ShapeDtypeStruct((NC, NP, DH), jnp.float32),
    mesh=_SC_MESH,
    scratch_types=[
        pltpu.VMEM_SHARED((NP, DH), jnp.float32),
        pltpu.VMEM_SHARED((NP, DH), jnp.float32),
        pltpu.VMEM((CPW, CHUNK), jnp.int32),
        pltpu.VMEM((CPW, CHUNK), jnp.int32),
        pltpu.VMEM((CHUNK, DH), jnp.float32),
        pltpu.VMEM((CHUNK, DH), jnp.float32),
        pltpu.VMEM((CHUNK, DH), jnp.float32),
        pltpu.VMEM((CHUNK, DH), jnp.float32),
        pltpu.SemaphoreType.DMA,
        pltpu.SemaphoreType.DMA,
    ],
    compiler_params=_SC_PARAMS,
)


def _dis_of(deg_ref):
    d = deg_ref[0, :, 0] + deg_ref[1, :, 0] + 1.0
    return lax.rsqrt(d)[:, None]


def _pre_body(x_ref, ga_ref, be_ref, w_ref, deg_ref, g_out):
    x = x_ref[...]
    mu = jnp.mean(x, axis=1, keepdims=True)
    xc = x - mu
    var = jnp.mean(xc * xc, axis=1, keepdims=True)
    ln = xc * lax.rsqrt(var + 1e-5) * ga_ref[...] + be_ref[...]
    u = jnp.dot(ln, w_ref[...], preferred_element_type=jnp.float32)
    g = u * _dis_of(deg_ref)
    g_out[0] = g[:, :DH]
    g_out[1] = g[:, DH:]


def _mid_body(acc_ref, g_ref, deg_ref, b_ref, w_ref, g_out):
    dis = _dis_of(deg_ref)
    h0 = jnp.maximum(dis * (acc_ref[0] + g_ref[0]) + b_ref[0], 0.0)
    h1 = jnp.maximum(dis * (acc_ref[1] + g_ref[1]) + b_ref[1], 0.0)
    u = (jnp.dot(h0, w_ref[0], preferred_element_type=jnp.float32)
         + jnp.dot(h1, w_ref[1], preferred_element_type=jnp.float32))
    g = u * dis
    g_out[0] = g[:, :DH]
    g_out[1] = g[:, DH:]


def _post_body(acc_ref, g_ref, deg_ref, b_ref, wo_ref, bo_ref, out_ref, h_out):
    dis = _dis_of(deg_ref)
    h0 = dis * (acc_ref[0] + g_ref[0]) + b_ref[0]
    h1 = dis * (acc_ref[1] + g_ref[1]) + b_ref[1]
    h_out[...] = jnp.concatenate([h0, h1], axis=1)
    out_ref[...] = (jnp.dot(h0, wo_ref[0], preferred_element_type=jnp.float32)
                    + jnp.dot(h1, wo_ref[1], preferred_element_type=jnp.float32)
                    + bo_ref[...])


_ROWS = pl.BlockSpec((RB, DD), lambda i: (i, 0))
_HALVES = pl.BlockSpec((NC, RB, DH), lambda i: (0, i, 0))
_BVEC = pl.BlockSpec((NC, 1, DH), lambda i: (0, 0, 0))
_VEC = pl.BlockSpec((1, DD), lambda i: (0, 0))
_MAT = pl.BlockSpec((DD, DD), lambda i: (0, 0))
_WSPLIT = pl.BlockSpec((NC, DH, DD), lambda i: (0, 0, 0))
_DEG = pl.BlockSpec((NC, RB, DEGW), lambda i: (0, i, 0))
_GRID = (NN // RB,)

_tc_pre = pl.pallas_call(
    _pre_body,
    grid=_GRID,
    in_specs=[_ROWS, _VEC, _VEC, _MAT, _DEG],
    out_specs=_HALVES,
    out_shape=jax.ShapeDtypeStruct((NC, NN, DH), jnp.float32),
)

_tc_mid = pl.pallas_call(
    _mid_body,
    grid=_GRID,
    in_specs=[_HALVES, _HALVES, _DEG, _BVEC, _WSPLIT],
    out_specs=_HALVES,
    out_shape=jax.ShapeDtypeStruct((NC, NN, DH), jnp.float32),
)

_tc_post = pl.pallas_call(
    _post_body,
    grid=_GRID,
    in_specs=[_HALVES, _HALVES, _DEG, _BVEC,
              pl.BlockSpec((NC, DH, DOUT), lambda i: (0, 0, 0)),
              pl.BlockSpec((1, DOUT), lambda i: (0, 0))],
    out_specs=[pl.BlockSpec((RB, DOUT), lambda i: (i, 0)), _ROWS],
    out_shape=[jax.ShapeDtypeStruct((NN, DOUT), jnp.float32),
               jax.ShapeDtypeStruct((NN, DD), jnp.float32)],
)


def kernel(x, edge_index, ln_gamma, ln_beta, W1, b1, Wh, bh, W2, b2, Wout, bout):
    ep = NS * CPT * CHUNK - EE
    pad = jnp.stack([jnp.zeros((ep,), jnp.int32),
                     jnp.full((ep,), NN, jnp.int32)])
    ei = jnp.concatenate([edge_index, pad], axis=1)
    src = ei[0].reshape(NS, CPT, CHUNK)
    dst = ei[1].reshape(NS, CPT, CHUNK)

    gamma = ln_gamma.reshape(1, DD)
    beta = ln_beta.reshape(1, DD)

    def bsplit(b):
        return b.reshape(NC, 1, DH)

    def wsplit(w):
        return w.reshape(NC, DH, w.shape[1])

    deg = _sc_degree(dst)
    g_a = _tc_pre(x, gamma, beta, W1, deg)
    acc_a = _sc_aggregate(g_a, src, dst)
    g_b = _tc_mid(acc_a, g_a, deg, bsplit(b1), wsplit(Wh))
    acc_b = _sc_aggregate(g_b, src, dst)
    g_c = _tc_mid(acc_b, g_b, deg, bsplit(bh), wsplit(W2))
    acc_c = _sc_aggregate(g_c, src, dst)
    out, h = _tc_post(acc_c, g_c, deg, bsplit(b2),
                      wsplit(Wout), bout.reshape(1, DOUT))
    return (out, h)


# speedup vs baseline: 21.5520x; 1.0138x over previous
import functools

import jax
import jax.numpy as jnp
from jax import lax
from jax.experimental import pallas as pl
from jax.experimental.pallas import tpu as pltpu
from jax.experimental.pallas import tpu_sc as plsc

NN = 10000
DD = 128
DH = 64
DOUT = 64
EE = 320000

NC = 2
NS = 16
CHUNK = 128
CPT = 160
CPW = 32
NP = 10112
RPT = NP // NS
DEGW = 16
RB = 2000

_SC_PARAMS = pltpu.CompilerParams(use_tc_tiling_on_sc=False)
_SC_MESH = plsc.VectorSubcoreMesh(core_axis_name="c", subcore_axis_name="s",
                                  num_cores=NC, num_subcores=NS)


def _fill(buf, rows, width, value):
    def row(r, _):
        for c in range(width // 16):
            buf[r, pl.ds(c * 16, 16)] = jnp.full((16,), value, jnp.float32)
        return 0
    lax.fori_loop(0, rows, row, 0)


def _spmem_zero_init(shared, buf, sid, width):
    _fill(buf, 128, width, 0.0)
    base = sid * RPT
    for k in range(4):
        pltpu.sync_copy(buf, shared.at[pl.ds(base + 128 * k, 128)])
    pltpu.sync_copy(buf.at[pl.ds(0, RPT - 512)],
                    shared.at[pl.ds(base + 512, RPT - 512)])


def _sc_degree_body(dst_hbm, out_hbm, deg_sh, dst_v, ones_v):
    c = lax.axis_index("c")
    s = lax.axis_index("s")
    _spmem_zero_init(deg_sh, ones_v, s, DEGW)
    _fill(ones_v, CHUNK, DEGW, 1.0)
    pltpu.sync_copy(dst_hbm.at[s].at[pl.ds(c * (CPT // 2), CPT // 2)], dst_v)
    plsc.subcore_barrier()

    def body(i, _):
        pltpu.sync_copy(ones_v, deg_sh.at[dst_v.at[i]], add=True)
        return 0
    lax.fori_loop(0, CPT // 2, body, 0)

    plsc.subcore_barrier()
    pltpu.sync_copy(deg_sh.at[pl.ds(s * RPT, RPT)],
                    out_hbm.at[c].at[pl.ds(s * RPT, RPT)])


def _sc_aggregate_body(g_hbm, src_hbm, dst_hbm, out_hbm,
                       g_sh, acc_sh, src_v0, dst_v0, src_v1, dst_v1,
                       buf0, buf1, buf2, buf3, gsem, ssem, isem):
    c = lax.axis_index("c")
    s = lax.axis_index("s")
    pltpu.async_copy(src_hbm.at[s].at[pl.ds(0, CPW)], src_v0, isem)
    pltpu.async_copy(dst_hbm.at[s].at[pl.ds(0, CPW)], dst_v0, isem)
    _spmem_zero_init(acc_sh, buf0, s, DH)
    start = jnp.minimum(s * RPT, NN - RPT)
    pltpu.sync_copy(g_hbm.at[c].at[pl.ds(start, RPT)],
                    g_sh.at[pl.ds(start, RPT)])
    plsc.subcore_barrier()

    bufs = (buf0, buf1, buf2, buf3)
    wins = ((src_v0, dst_v0), (src_v1, dst_v1))
    for p in range(CPT // CPW):
        src_v, dst_v = wins[p % 2]
        pltpu.make_async_copy(src_hbm.at[s].at[pl.ds(p * CPW, CPW)],
                              src_v, isem).wait()
        pltpu.make_async_copy(dst_hbm.at[s].at[pl.ds(p * CPW, CPW)],
                              dst_v, isem).wait()
        if p + 1 < CPT // CPW:
            sn, dn = wins[(p + 1) % 2]
            pltpu.async_copy(src_hbm.at[s].at[pl.ds((p + 1) * CPW, CPW)],
                             sn, isem)
            pltpu.async_copy(dst_hbm.at[s].at[pl.ds((p + 1) * CPW, CPW)],
                             dn, isem)
        pltpu.async_copy(g_sh.at[src_v.at[0]], bufs[0], gsem)
        pltpu.async_copy(g_sh.at[src_v.at[1]], bufs[1], gsem)

        def body(i, _):
            for j in range(4):
                q = 4 * i + j
                pltpu.make_async_copy(g_sh.at[src_v.at[q]], bufs[j],
                                      gsem).wait()
                pltpu.async_copy(bufs[j], acc_sh.at[dst_v.at[q]], ssem,
                                 add=True)
                if j < 2:
                    @pl.when(i >= 1)
                    def _():
                        pltpu.make_async_copy(bufs[0], acc_sh.at[dst_v.at[0]],
                                              ssem).wait()
                else:
                    pltpu.make_async_copy(bufs[0], acc_sh.at[dst_v.at[0]],
                                          ssem).wait()
                jn = (j + 2) % 4
                if j < 2:
                    pltpu.async_copy(g_sh.at[src_v.at[q + 2]], bufs[jn], gsem)
                else:
                    @pl.when(i < CPW // 4 - 1)
                    def _():
                        pltpu.async_copy(g_sh.at[src_v.at[q + 2]], bufs[jn],
                                         gsem)
            return 0
        lax.fori_loop(0, CPW // 4, body, 0)
        for _ in range(2):
            pltpu.make_async_copy(bufs[0], acc_sh.at[dst_v.at[0]],
                                  ssem).wait()

    plsc.subcore_barrier()
    pltpu.sync_copy(acc_sh.at[pl.ds(s * RPT, RPT)],
                    out_hbm.at[c].at[pl.ds(s * RPT, RPT)])


_sc_degree = pl.kernel(
    _sc_degree_body,
    out_type=jax.ShapeDtypeStruct((NC, NP, DEGW), jnp.float32),
    mesh=_SC_MESH,
    scratch_types=[
        pltpu.VMEM_SHARED((NP, DEGW), jnp.float32),
        pltpu.VMEM((CPT // 2, CHUNK), jnp.int32),
        pltpu.VMEM((CHUNK, DEGW), jnp.float32),
    ],
    compiler_params=_SC_PARAMS,
)

_sc_aggregate = pl.kernel(
    _sc_aggregate_body,
    out_type=jax.ShapeDtypeStruct((NC, NP, DH), jnp.float32),
    mesh=_SC_MESH,
    scratch_types=[
        pltpu.VMEM_SHARED((NP, DH), jnp.float32),
        pltpu.VMEM_SHARED((NP, DH), jnp.float32),
        pltpu.VMEM((CPW, CHUNK), jnp.int32),
        pltpu.VMEM((CPW, CHUNK), jnp.int32),
        pltpu.VMEM((CPW, CHUNK), jnp.int32),
        pltpu.VMEM((CPW, CHUNK), jnp.int32),
        pltpu.VMEM((CHUNK, DH), jnp.float32),
        pltpu.VMEM((CHUNK, DH), jnp.float32),
        pltpu.VMEM((CHUNK, DH), jnp.float32),
        pltpu.VMEM((CHUNK, DH), jnp.float32),
        pltpu.SemaphoreType.DMA,
        pltpu.SemaphoreType.DMA,
        pltpu.SemaphoreType.DMA,
    ],
    compiler_params=_SC_PARAMS,
)


def _dis_of(deg_ref):
    d = deg_ref[0, :, 0] + deg_ref[1, :, 0] + 1.0
    return lax.rsqrt(d)[:, None]


def _pre_body(x_ref, ga_ref, be_ref, w_ref, deg_ref, g_out):
    x = x_ref[...]
    mu = jnp.mean(x, axis=1, keepdims=True)
    xc = x - mu
    var = jnp.mean(xc * xc, axis=1, keepdims=True)
    ln = xc * lax.rsqrt(var + 1e-5) * ga_ref[...] + be_ref[...]
    u = jnp.dot(ln, w_ref[...], preferred_element_type=jnp.float32)
    g = u * _dis_of(deg_ref)
    g_out[0] = g[:, :DH]
    g_out[1] = g[:, DH:]


def _mid_body(acc_ref, g_ref, deg_ref, b_ref, w_ref, g_out):
    dis = _dis_of(deg_ref)
    h0 = jnp.maximum(dis * (acc_ref[0] + g_ref[0]) + b_ref[0], 0.0)
    h1 = jnp.maximum(dis * (acc_ref[1] + g_ref[1]) + b_ref[1], 0.0)
    u = (jnp.dot(h0, w_ref[0], preferred_element_type=jnp.float32)
         + jnp.dot(h1, w_ref[1], preferred_element_type=jnp.float32))
    g = u * dis
    g_out[0] = g[:, :DH]
    g_out[1] = g[:, DH:]


def _post_body(acc_ref, g_ref, deg_ref, b_ref, wo_ref, bo_ref, out_ref, h_out):
    dis = _dis_of(deg_ref)
    h0 = dis * (acc_ref[0] + g_ref[0]) + b_ref[0]
    h1 = dis * (acc_ref[1] + g_ref[1]) + b_ref[1]
    h_out[...] = jnp.concatenate([h0, h1], axis=1)
    out_ref[...] = (jnp.dot(h0, wo_ref[0], preferred_element_type=jnp.float32)
                    + jnp.dot(h1, wo_ref[1], preferred_element_type=jnp.float32)
                    + bo_ref[...])


_ROWS = pl.BlockSpec((RB, DD), lambda i: (i, 0))
_HALVES = pl.BlockSpec((NC, RB, DH), lambda i: (0, i, 0))
_BVEC = pl.BlockSpec((NC, 1, DH), lambda i: (0, 0, 0))
_VEC = pl.BlockSpec((1, DD), lambda i: (0, 0))
_MAT = pl.BlockSpec((DD, DD), lambda i: (0, 0))
_WSPLIT = pl.BlockSpec((NC, DH, DD), lambda i: (0, 0, 0))
_DEG = pl.BlockSpec((NC, RB, DEGW), lambda i: (0, i, 0))
_GRID = (NN // RB,)

_tc_pre = pl.pallas_call(
    _pre_body,
    grid=_GRID,
    in_specs=[_ROWS, _VEC, _VEC, _MAT, _DEG],
    out_specs=_HALVES,
    out_shape=jax.ShapeDtypeStruct((NC, NN, DH), jnp.float32),
)

_tc_mid = pl.pallas_call(
    _mid_body,
    grid=_GRID,
    in_specs=[_HALVES, _HALVES, _DEG, _BVEC, _WSPLIT],
    out_specs=_HALVES,
    out_shape=jax.ShapeDtypeStruct((NC, NN, DH), jnp.float32),
)

_tc_post = pl.pallas_call(
    _post_body,
    grid=_GRID,
    in_specs=[_HALVES, _HALVES, _DEG, _BVEC,
              pl.BlockSpec((NC, DH, DOUT), lambda i: (0, 0, 0)),
              pl.BlockSpec((1, DOUT), lambda i: (0, 0))],
    out_specs=[pl.BlockSpec((RB, DOUT), lambda i: (i, 0)), _ROWS],
    out_shape=[jax.ShapeDtypeStruct((NN, DOUT), jnp.float32),
               jax.ShapeDtypeStruct((NN, DD), jnp.float32)],
)


def kernel(x, edge_index, ln_gamma, ln_beta, W1, b1, Wh, bh, W2, b2, Wout, bout):
    ep = NS * CPT * CHUNK - EE
    pad = jnp.stack([jnp.zeros((ep,), jnp.int32),
                     jnp.full((ep,), NN, jnp.int32)])
    ei = jnp.concatenate([edge_index, pad], axis=1)
    src = ei[0].reshape(NS, CPT, CHUNK)
    dst = ei[1].reshape(NS, CPT, CHUNK)

    gamma = ln_gamma.reshape(1, DD)
    beta = ln_beta.reshape(1, DD)

    def bsplit(b):
        return b.reshape(NC, 1, DH)

    def wsplit(w):
        return w.reshape(NC, DH, w.shape[1])

    deg = _sc_degree(dst)
    g_a = _tc_pre(x, gamma, beta, W1, deg)
    acc_a = _sc_aggregate(g_a, src, dst)
    g_b = _tc_mid(acc_a, g_a, deg, bsplit(b1), wsplit(Wh))
    acc_b = _sc_aggregate(g_b, src, dst)
    g_c = _tc_mid(acc_b, g_b, deg, bsplit(bh), wsplit(W2))
    acc_c = _sc_aggregate(g_c, src, dst)
    out, h = _tc_post(acc_c, g_c, deg, bsplit(b2),
                      wsplit(Wout), bout.reshape(1, DOUT))
    return (out, h)
